# baseline (speedup 1.0000x reference)
import functools

import jax
import jax.numpy as jnp
from jax import lax
from jax.experimental import pallas as pl
from jax.experimental.pallas import tpu as pltpu
from jax.experimental.pallas import tpu_sc as plsc

_B, _S, _D = 4, 2048, 768
_NC, _NS = 2, 16
_NW = _NC * _NS
_PPW = _S // _NW
_CH = 8
_CPB = _PPW // _CH
_NBUF = 12
_LPR = _D // 16

_mesh = plsc.VectorSubcoreMesh(core_axis_name="c", subcore_axis_name="s")


@functools.partial(
    pl.kernel,
    mesh=_mesh,
    out_type=jax.ShapeDtypeStruct((_B, _S, _D), jnp.float32),
    scratch_types=[
        pltpu.VMEM((_B, _PPW), jnp.int32),
        pltpu.VMEM((_PPW, _D), jnp.float32),
        pltpu.VMEM((_NBUF, _CH, _D), jnp.float32),
        pltpu.SemaphoreType.DMA((3,)),
        pltpu.SemaphoreType.DMA((_NBUF,)),
        pltpu.SemaphoreType.DMA((_NBUF,)),
    ],
)
def _emb_kernel(x_hbm, wte_hbm, wpe_hbm, out_hbm,
                idx_v, wpe_v, tok_v,
                sem_in, gsems, osems):
    wid = lax.axis_index("s") * _NC + lax.axis_index("c")
    pos0 = wid * _PPW

    idx0_copy = pltpu.async_copy(
        x_hbm.at[0, pl.ds(pos0, _PPW)], idx_v.at[0], sem_in.at[0])
    idx_rest = [
        pltpu.async_copy(x_hbm.at[b, pl.ds(pos0, _PPW)], idx_v.at[b],
                         sem_in.at[1])
        for b in range(1, _B)
    ]
    wpe_copy = pltpu.async_copy(
        wpe_hbm.at[pl.ds(pos0, _PPW)], wpe_v, sem_in.at[2])

    _NCHUNK = _B * _CPB
    _PF = 8
    _DR = _NBUF - _PF

    def gather_desc(c):
        b, part, buf = c // _CPB, c % _CPB, c % _NBUF
        return pltpu.make_async_copy(
            wte_hbm.at[idx_v.at[b, pl.ds(part * _CH, _CH)]],
            tok_v.at[buf], gsems.at[buf])

    def store_desc(c):
        b, part, buf = c // _CPB, c % _CPB, c % _NBUF
        return pltpu.make_async_copy(
            tok_v.at[buf],
            out_hbm.at[b, pl.ds(pos0 + part * _CH, _CH)], osems.at[buf])

    def add_chunk(buf, part):
        @plsc.parallel_loop(0, _CH, unroll=2)
        def row_body(r):
            for g in range(_LPR // 8):
                w = [wpe_v[part * _CH + r, pl.ds((g * 8 + j) * 16, 16)]
                     for j in range(8)]
                for j in range(8):
                    plsc.addupdate(
                        tok_v.at[buf, r, pl.ds((g * 8 + j) * 16, 16)], w[j])

    idx0_copy.wait()
    for c in range(_PF):
        gather_desc(c).start()
    for c in idx_rest:
        c.wait()
    wpe_copy.wait()

    @pl.loop(0, _NCHUNK)
    def chunk_step(c):
        @pl.when(jnp.logical_and(c >= _DR, c < _NCHUNK - _PF))
        def _drain():
            store_desc(c - _DR).wait()

        @pl.when(c < _NCHUNK - _PF)
        def _prefetch():
            gather_desc(c + _PF).start()

        gather_desc(c).wait()
        add_chunk(c % _NBUF, c % _CPB)
        store_desc(c).start()

    for c in range(_NCHUNK - _NBUF, _NCHUNK):
        store_desc(c).wait()


def kernel(x, wte, wpe):
    return _emb_kernel(x.astype(jnp.int32), wte, wpe)


# speedup vs baseline: 1.0054x; 1.0054x over previous
import functools

import jax
import jax.numpy as jnp
from jax import lax
from jax.experimental import pallas as pl
from jax.experimental.pallas import tpu as pltpu
from jax.experimental.pallas import tpu_sc as plsc

_B, _S, _D = 4, 2048, 768
_NC, _NS = 2, 16
_NW = _NC * _NS
_PPW = _S // _NW
_CH = 16
_CPB = _PPW // _CH
_NBUF = 6
_LPR = _D // 16

_mesh = plsc.VectorSubcoreMesh(core_axis_name="c", subcore_axis_name="s")


@functools.partial(
    pl.kernel,
    mesh=_mesh,
    out_type=jax.ShapeDtypeStruct((_B, _S, _D), jnp.float32),
    scratch_types=[
        pltpu.VMEM((_B, _PPW), jnp.int32),
        pltpu.VMEM((_PPW, _D), jnp.float32),
        pltpu.VMEM((_NBUF, _CH, _D), jnp.float32),
        pltpu.SemaphoreType.DMA((3,)),
        pltpu.SemaphoreType.DMA((_NBUF,)),
        pltpu.SemaphoreType.DMA((_NBUF,)),
    ],
)
def _emb_kernel(x_hbm, wte_hbm, wpe_hbm, out_hbm,
                idx_v, wpe_v, tok_v,
                sem_in, gsems, osems):
    wid = lax.axis_index("s") * _NC + lax.axis_index("c")
    pos0 = wid * _PPW

    idx0_copy = pltpu.async_copy(
        x_hbm.at[0, pl.ds(pos0, _PPW)], idx_v.at[0], sem_in.at[0])
    idx_rest = [
        pltpu.async_copy(x_hbm.at[b, pl.ds(pos0, _PPW)], idx_v.at[b],
                         sem_in.at[1])
        for b in range(1, _B)
    ]
    wpe_copy = pltpu.async_copy(
        wpe_hbm.at[pl.ds(pos0, _PPW)], wpe_v, sem_in.at[2])

    _NCHUNK = _B * _CPB
    _PF = 4
    _DR = _NBUF - _PF

    def gather_desc(c):
        b, part, buf = c // _CPB, c % _CPB, c % _NBUF
        return pltpu.make_async_copy(
            wte_hbm.at[idx_v.at[b, pl.ds(part * _CH, _CH)]],
            tok_v.at[buf], gsems.at[buf])

    def store_desc(c):
        b, part, buf = c // _CPB, c % _CPB, c % _NBUF
        return pltpu.make_async_copy(
            tok_v.at[buf],
            out_hbm.at[b, pl.ds(pos0 + part * _CH, _CH)], osems.at[buf])

    def add_chunk(buf, part):
        @plsc.parallel_loop(0, _CH, unroll=2)
        def row_body(r):
            for g in range(_LPR // 8):
                w = [wpe_v[part * _CH + r, pl.ds((g * 8 + j) * 16, 16)]
                     for j in range(8)]
                for j in range(8):
                    plsc.addupdate(
                        tok_v.at[buf, r, pl.ds((g * 8 + j) * 16, 16)], w[j])

    idx0_copy.wait()
    for c in range(_PF):
        gather_desc(c).start()
    for c in idx_rest:
        c.wait()
    wpe_copy.wait()

    @pl.loop(0, _NCHUNK)
    def chunk_step(c):
        @pl.when(jnp.logical_and(c >= _DR, c < _NCHUNK - _PF))
        def _drain():
            store_desc(c - _DR).wait()

        @pl.when(c < _NCHUNK - _PF)
        def _prefetch():
            gather_desc(c + _PF).start()

        gather_desc(c).wait()
        add_chunk(c % _NBUF, c % _CPB)
        store_desc(c).start()

    for c in range(_NCHUNK - _NBUF, _NCHUNK):
        store_desc(c).wait()


def kernel(x, wte, wpe):
    return _emb_kernel(x.astype(jnp.int32), wte, wpe)


# speedup vs baseline: 1.0061x; 1.0007x over previous
import functools

import jax
import jax.numpy as jnp
from jax import lax
from jax.experimental import pallas as pl
from jax.experimental.pallas import tpu as pltpu
from jax.experimental.pallas import tpu_sc as plsc

_B, _S, _D = 4, 2048, 768
_NC, _NS = 2, 16
_NW = _NC * _NS
_PPW = _S // _NW
_CH = 16
_CPB = _PPW // _CH
_NBUF = 6
_LPR = _D // 16

_mesh = plsc.VectorSubcoreMesh(core_axis_name="c", subcore_axis_name="s")


@functools.partial(
    pl.kernel,
    mesh=_mesh,
    out_type=jax.ShapeDtypeStruct((_B, _S, _D), jnp.float32),
    scratch_types=[
        pltpu.VMEM((_B, _PPW), jnp.int32),
        pltpu.VMEM((_PPW, _D), jnp.float32),
        pltpu.VMEM((_NBUF, _CH, _D), jnp.float32),
        pltpu.SemaphoreType.DMA((2,)),
        pltpu.SemaphoreType.DMA((_CPB,)),
        pltpu.SemaphoreType.DMA((_NBUF,)),
        pltpu.SemaphoreType.DMA((_NBUF,)),
    ],
)
def _emb_kernel(x_hbm, wte_hbm, wpe_hbm, out_hbm,
                idx_v, wpe_v, tok_v,
                sem_in, sem_wpe, gsems, osems):
    wid = lax.axis_index("s") * _NC + lax.axis_index("c")
    pos0 = wid * _PPW

    _NCHUNK = _B * _CPB
    _PF = 4
    _DR = _NBUF - _PF

    def gather_desc(c):
        b, part, buf = c // _CPB, c % _CPB, c % _NBUF
        return pltpu.make_async_copy(
            wte_hbm.at[idx_v.at[b, pl.ds(part * _CH, _CH)]],
            tok_v.at[buf], gsems.at[buf])

    def store_desc(c):
        b, part, buf = c // _CPB, c % _CPB, c % _NBUF
        return pltpu.make_async_copy(
            tok_v.at[buf],
            out_hbm.at[b, pl.ds(pos0 + part * _CH, _CH)], osems.at[buf])

    def add_chunk(buf, part):
        @plsc.parallel_loop(0, _CH, unroll=2)
        def row_body(r):
            for g in range(_LPR // 8):
                w = [wpe_v[part * _CH + r, pl.ds((g * 8 + j) * 16, 16)]
                     for j in range(8)]
                for j in range(8):
                    plsc.addupdate(
                        tok_v.at[buf, r, pl.ds((g * 8 + j) * 16, 16)], w[j])

    def wpe_part_desc(p):
        return pltpu.make_async_copy(
            wpe_hbm.at[pl.ds(pos0 + p * _CH, _CH)],
            wpe_v.at[pl.ds(p * _CH, _CH)], sem_wpe.at[p])

    idx0_copy = pltpu.async_copy(
        x_hbm.at[0, pl.ds(pos0, _PPW)], idx_v.at[0], sem_in.at[0])
    idx0_copy.wait()
    for c in range(_PF):
        gather_desc(c).start()
        wpe_part_desc(c).start()
    idx_rest = [
        pltpu.async_copy(x_hbm.at[b, pl.ds(pos0, _PPW)], idx_v.at[b],
                         sem_in.at[1])
        for b in range(1, _B)
    ]
    for c in idx_rest:
        c.wait()

    @pl.loop(0, _NCHUNK)
    def chunk_step(c):
        @pl.when(jnp.logical_and(c >= _DR, c < _NCHUNK - _PF))
        def _drain():
            store_desc(c - _DR).wait()

        @pl.when(c < _NCHUNK - _PF)
        def _prefetch():
            gather_desc(c + _PF).start()

        gather_desc(c).wait()

        @pl.when(c < _CPB)
        def _wpe_ready():
            wpe_part_desc(c).wait()

        add_chunk(c % _NBUF, c % _CPB)
        store_desc(c).start()

    for c in range(_NCHUNK - _NBUF, _NCHUNK):
        store_desc(c).wait()


def kernel(x, wte, wpe):
    return _emb_kernel(x.astype(jnp.int32), wte, wpe)
